# baseline (speedup 1.0000x reference)
import functools

import jax
import jax.numpy as jnp
from jax import lax
from jax.experimental import pallas as pl
from jax.experimental.pallas import tpu as pltpu
from jax.experimental.pallas import tpu_sc as plsc

_B = 4
_T = 2048
_D = 768
_BT = _B * _T
_NC = 2
_NS = 16
_NW = _NC * _NS
_PPW = _T // _NW
_CH = 32
_SUB = _PPW // _CH
_NV = _D // 16


def _emb_kernel(idx_hbm, tok_hbm, pos_hbm, out_hbm, idx_v, pos_v, tok_v, sem):
    wid = lax.axis_index("s") * _NC + lax.axis_index("c")
    pos_base = wid * _PPW
    pltpu.sync_copy(pos_hbm.at[pl.ds(pos_base, _PPW)], pos_v)

    def chunk_body(c, _):
        b = c // _SUB
        s = c % _SUB
        flat_base = b * _T + pos_base + s * _CH
        pltpu.sync_copy(idx_hbm.at[pl.ds(flat_base, _CH)], idx_v)
        pltpu.async_copy(tok_hbm.at[idx_v], tok_v, sem).wait()

        def row_body(j, _):
            p = s * _CH + j
            for k in range(_NV):
                col = k * 16
                plsc.addupdate(
                    tok_v.at[j, pl.ds(col, 16)], pos_v[p, pl.ds(col, 16)]
                )
            return ()

        lax.fori_loop(0, _CH, row_body, ())
        pltpu.sync_copy(tok_v, out_hbm.at[pl.ds(flat_base, _CH)])
        return ()

    lax.fori_loop(0, _B * _SUB, chunk_body, ())


def kernel(idx, tok_weight, pos_weight):
    idx_flat = idx.reshape(_BT).astype(jnp.int32)
    mesh = plsc.VectorSubcoreMesh(core_axis_name="c", subcore_axis_name="s")
    run = functools.partial(
        pl.kernel,
        out_type=jax.ShapeDtypeStruct((_BT, _D), jnp.float32),
        mesh=mesh,
        scratch_types=[
            pltpu.VMEM((_CH,), jnp.int32),
            pltpu.VMEM((_PPW, _D), jnp.float32),
            pltpu.VMEM((_CH, _D), jnp.float32),
            pltpu.SemaphoreType.DMA,
        ],
    )(_emb_kernel)
    out = run(idx_flat, tok_weight, pos_weight)
    return out.reshape(_B, _T, _D)


# speedup vs baseline: 1.5127x; 1.5127x over previous
import functools

import jax
import jax.numpy as jnp
from jax import lax
from jax.experimental import pallas as pl
from jax.experimental.pallas import tpu as pltpu
from jax.experimental.pallas import tpu_sc as plsc

_B = 4
_T = 2048
_D = 768
_BT = _B * _T
_NC = 2
_NS = 16
_NW = _NC * _NS
_PPW = _T // _NW
_CH = 32
_SUB = _PPW // _CH
_NCH = _B * _SUB
_NV = _D // 16
_NBUF = 3


def _emb_kernel(idx_hbm, tok_hbm, pos_hbm, out_hbm,
                idx_v, pos_v, bufs, isem, psem, gsems, osems):
    wid = lax.axis_index("s") * _NC + lax.axis_index("c")
    pos_base = wid * _PPW

    pos_d = pltpu.async_copy(pos_hbm.at[pl.ds(pos_base, _PPW)], pos_v, psem)
    idx_d = [
        pltpu.async_copy(
            idx_hbm.at[pl.ds(b * _T + pos_base, _PPW)], idx_v.at[b], isem
        )
        for b in range(_B)
    ]

    def add_rows(buf, s):
        def row_body(j, _):
            p = s * _CH + j
            for k in range(_NV):
                col = k * 16
                plsc.addupdate(
                    buf.at[j, pl.ds(col, 16)], pos_v[p, pl.ds(col, 16)]
                )
            return ()

        lax.fori_loop(0, _CH, row_body, ())

    def start_gather(c):
        b, s = c // _SUB, c % _SUB
        if c < _B:
            idx_d[c].wait()
        return pltpu.async_copy(
            tok_hbm.at[idx_v.at[b, pl.ds(s * _CH, _CH)]],
            bufs[c % _NBUF],
            gsems[c % _NBUF],
        )

    gd = [None] * _NCH
    od = [None] * _NCH
    gd[0] = start_gather(0)
    gd[1] = start_gather(1)
    pos_d.wait()
    for c in range(_NCH):
        p = c % _NBUF
        gd[c].wait()
        if c + 2 < _NCH:
            if c - 1 >= 0:
                od[c - 1].wait()
            gd[c + 2] = start_gather(c + 2)
        b, s = c // _SUB, c % _SUB
        add_rows(bufs[p], s)
        od[c] = pltpu.async_copy(
            bufs[p],
            out_hbm.at[pl.ds(b * _T + pos_base + s * _CH, _CH)],
            osems[p],
        )
    od[_NCH - 2].wait()
    od[_NCH - 1].wait()


def kernel(idx, tok_weight, pos_weight):
    idx_flat = idx.reshape(_BT).astype(jnp.int32)
    mesh = plsc.VectorSubcoreMesh(core_axis_name="c", subcore_axis_name="s")
    run = functools.partial(
        pl.kernel,
        out_type=jax.ShapeDtypeStruct((_BT, _D), jnp.float32),
        mesh=mesh,
        scratch_types=[
            pltpu.VMEM((_B, _PPW), jnp.int32),
            pltpu.VMEM((_PPW, _D), jnp.float32),
            [pltpu.VMEM((_CH, _D), jnp.float32) for _ in range(_NBUF)],
            pltpu.SemaphoreType.DMA,
            pltpu.SemaphoreType.DMA,
            [pltpu.SemaphoreType.DMA for _ in range(_NBUF)],
            [pltpu.SemaphoreType.DMA for _ in range(_NBUF)],
        ],
    )(_emb_kernel)
    out = run(idx_flat, tok_weight, pos_weight)
    return out.reshape(_B, _T, _D)
